# baseline (speedup 1.0000x reference)
import functools

import jax
import jax.numpy as jnp
from jax import lax
from jax.experimental import pallas as pl
from jax.experimental.pallas import tpu as pltpu
from jax.experimental.pallas import tpu_sc as plsc

_B, _S, _D, _E, _TOPK = 4, 2048, 2048, 16, 2


def _logits_block(x_ref, w_ref, b_ref, n_ref, noisy_ref):
    y = jnp.dot(x_ref[:, :], w_ref[:, :], preferred_element_type=jnp.float32)
    yt = y.T + b_ref[:, :]
    logits = yt[:_E, :]
    noise_logits = yt[_E:, :]
    sp = jnp.maximum(noise_logits, 0.0) + jnp.log1p(
        jnp.exp(-jnp.abs(noise_logits)))
    noisy_ref[:, :] = logits + n_ref[:, :] * sp


def _tc_noisy_logits(x, w, b, noise_t, n_tokens):
    T = 1024
    nb = n_tokens // T
    return pl.pallas_call(
        _logits_block,
        grid=(nb,),
        in_specs=[
            pl.BlockSpec((T, _D), lambda i: (i, 0)),
            pl.BlockSpec((_D, 2 * _E), lambda i: (0, 0)),
            pl.BlockSpec((2 * _E, 1), lambda i: (0, 0)),
            pl.BlockSpec((_E, T), lambda i: (0, i)),
        ],
        out_specs=pl.BlockSpec((_E, T), lambda i: (0, i)),
        out_shape=jax.ShapeDtypeStruct((_E, n_tokens), jnp.float32),
        compiler_params=pltpu.CompilerParams(
            dimension_semantics=("arbitrary",),
        ),
    )(x, w, b, noise_t)


_NW = 32
_L = 16


def _sc_route(noisy_t, n_tokens):
    tpw = n_tokens // _NW
    ng = tpw // _L
    mesh = plsc.VectorSubcoreMesh(core_axis_name="c", subcore_axis_name="s")

    @functools.partial(
        pl.kernel,
        mesh=mesh,
        out_type=[
            jax.ShapeDtypeStruct((_E, n_tokens), jnp.float32),
            jax.ShapeDtypeStruct((_TOPK, n_tokens), jnp.int32),
            jax.ShapeDtypeStruct((_NW, 2 * _E, _L), jnp.float32),
        ],
        scratch_types=[
            pltpu.VMEM((_E, tpw), jnp.float32),
            pltpu.VMEM((_E, tpw), jnp.float32),
            pltpu.VMEM((_TOPK, tpw), jnp.int32),
            pltpu.VMEM((2 * _E, _L), jnp.float32),
        ],
    )
    def route(noisy_hbm, probs_hbm, idx_hbm, part_hbm,
              in_v, probs_v, idx_v, part_v):
        wid = lax.axis_index("s") * 2 + lax.axis_index("c")
        base = wid * tpw
        pltpu.sync_copy(noisy_hbm.at[:, pl.ds(base, tpw)], in_v)

        ninf = jnp.float32(float("-inf"))
        zf = jnp.zeros((_L,), jnp.float32)

        def body(g, carry):
            acc_p, acc_m = carry
            col = g * _L
            v = [in_v[e, pl.ds(col, _L)] for e in range(_E)]
            m1 = v[0]
            for e in range(1, _E):
                m1 = jnp.maximum(m1, v[e])
            i1 = jnp.where(v[0] == m1, 0, _E)
            for e in range(1, _E):
                i1 = jnp.minimum(i1, jnp.where(v[e] == m1, e, _E))
            m2 = jnp.where(i1 == 0, ninf, v[0])
            for e in range(1, _E):
                m2 = jnp.maximum(m2, jnp.where(i1 == e, ninf, v[e]))
            i2 = jnp.where((v[0] == m2) & (i1 != 0), 0, _E)
            for e in range(1, _E):
                i2 = jnp.minimum(
                    i2, jnp.where((v[e] == m2) & (i1 != e), e, _E))
            s = zf
            p = []
            for e in range(_E):
                hit = (i1 == e) | (i2 == e)
                pe = jnp.where(hit, jnp.exp(v[e] - m1), 0.0)
                p.append(pe)
                s = s + pe
                acc_m = tuple(
                    am + jnp.where(hit, 1.0, 0.0) if j == e else am
                    for j, am in enumerate(acc_m))
            rinv = 1.0 / s
            new_p = []
            for e in range(_E):
                pr = p[e] * rinv
                probs_v[e, pl.ds(col, _L)] = pr
                new_p.append(acc_p[e] + pr)
            idx_v[0, pl.ds(col, _L)] = i1
            idx_v[1, pl.ds(col, _L)] = i2
            return (tuple(new_p), acc_m)

        zeros = tuple(zf for _ in range(_E))
        acc_p, acc_m = lax.fori_loop(0, ng, body, (zeros, zeros))
        for e in range(_E):
            part_v[e] = acc_p[e]
            part_v[_E + e] = acc_m[e]
        pltpu.sync_copy(probs_v, probs_hbm.at[:, pl.ds(base, tpw)])
        pltpu.sync_copy(idx_v, idx_hbm.at[:, pl.ds(base, tpw)])
        pltpu.sync_copy(part_v, part_hbm.at[wid])

    return route(noisy_t)


def kernel(mh_output, W_route, b_route, W_noise, b_noise):
    n_tokens = _B * _S
    x = mh_output.reshape(n_tokens, _D)
    w = jnp.concatenate([W_route, W_noise], axis=1)
    b = jnp.concatenate([b_route, b_noise], axis=0).reshape(2 * _E, 1)
    with jax.ensure_compile_time_eval():
        noise = jax.random.normal(jax.random.key(42), (_B, _S, _E),
                                  dtype=jnp.float32)
        noise_t = noise.reshape(n_tokens, _E).T.copy()

    noisy_t = _tc_noisy_logits(x, w, b, noise_t, n_tokens)
    probs_t, idx_t, part = _sc_route(noisy_t, n_tokens)

    sum_p = jnp.sum(part[:, :_E, :], axis=(0, 2))
    sum_m = jnp.sum(part[:, _E:, :], axis=(0, 2))
    scale = jnp.float32(_E) / jnp.float32(n_tokens * n_tokens)
    loss = scale * jnp.sum(sum_p * sum_m)

    return (probs_t.T.reshape(_B, _S, _E),
            idx_t.T.reshape(_B, _S, _TOPK),
            loss)


# speedup vs baseline: 1.6337x; 1.6337x over previous
import functools

import jax
import jax.numpy as jnp
from jax.experimental import pallas as pl
from jax.experimental.pallas import tpu as pltpu

_B, _S, _D, _E, _TOPK = 4, 2048, 2048, 16, 2


def _router_block(xa_ref, xb_ref, w_ref, b_ref, n_ref, probs_ref, idx_ref,
                  loss_ref, acc_ref, *, n_tokens):
    i = pl.program_id(0)
    nb = pl.num_programs(0)
    kh = _D // 2
    y = (jnp.dot(xa_ref[:, :], w_ref[:kh, :], preferred_element_type=jnp.float32)
         + jnp.dot(xb_ref[:, :], w_ref[kh:, :], preferred_element_type=jnp.float32))
    yt = y.T + b_ref[:, :]
    logits = yt[:_E, :]
    noise_logits = yt[_E:, :]
    sp = jnp.maximum(noise_logits, 0.0) + jnp.log1p(jnp.exp(-jnp.abs(noise_logits)))
    noisy = logits + n_ref[:, :] * sp

    iota = jax.lax.broadcasted_iota(jnp.int32, noisy.shape, 0)
    m1 = jnp.max(noisy, axis=0, keepdims=True)
    i1 = jnp.min(jnp.where(noisy == m1, iota, _E), axis=0, keepdims=True)
    masked = jnp.where(iota == i1, -jnp.inf, noisy)
    m2 = jnp.max(masked, axis=0, keepdims=True)
    i2 = jnp.min(jnp.where(masked == m2, iota, _E), axis=0, keepdims=True)

    mask = (iota == i1) | (iota == i2)
    p = jnp.where(mask, jnp.exp(noisy - m1), 0.0)
    probs = p / jnp.sum(p, axis=0, keepdims=True)
    probs_ref[:, :] = probs
    idx_ref[:, :] = jnp.concatenate([i1, i2], axis=0)

    bp = jnp.sum(probs, axis=1, keepdims=True)
    bm = jnp.sum(mask.astype(jnp.float32), axis=1, keepdims=True)

    @pl.when(i == 0)
    def _():
        acc_ref[:, 0:1] = bp
        acc_ref[:, 1:2] = bm

    @pl.when(i > 0)
    def _():
        acc_ref[:, 0:1] = acc_ref[:, 0:1] + bp
        acc_ref[:, 1:2] = acc_ref[:, 1:2] + bm

    @pl.when(i == nb - 1)
    def _():
        scale = jnp.float32(_E) / jnp.float32(n_tokens * n_tokens)
        loss = scale * jnp.sum(acc_ref[:, 0:1] * acc_ref[:, 1:2],
                               axis=0, keepdims=True)
        loss_ref[:, :] = loss


def kernel(mh_output, W_route, b_route, W_noise, b_noise):
    n_tokens = _B * _S
    x = mh_output.reshape(n_tokens, _D)
    w = jnp.concatenate([W_route, W_noise], axis=1)
    b = jnp.concatenate([b_route, b_noise], axis=0).reshape(2 * _E, 1)
    with jax.ensure_compile_time_eval():
        noise = jax.random.normal(jax.random.key(42), (_B, _S, _E),
                                  dtype=jnp.float32)
        noise_t = noise.reshape(n_tokens, _E).T.copy()

    T = 1024
    nb = n_tokens // T
    probs_t, idx_t, loss = pl.pallas_call(
        functools.partial(_router_block, n_tokens=n_tokens),
        grid=(nb,),
        in_specs=[
            pl.BlockSpec((T, _D // 2), lambda i: (i, 0)),
            pl.BlockSpec((T, _D // 2), lambda i: (i, 1)),
            pl.BlockSpec((_D, 2 * _E), lambda i: (0, 0)),
            pl.BlockSpec((2 * _E, 1), lambda i: (0, 0)),
            pl.BlockSpec((_E, T), lambda i: (0, i)),
        ],
        out_specs=[
            pl.BlockSpec((_E, T), lambda i: (0, i)),
            pl.BlockSpec((_TOPK, T), lambda i: (0, i)),
            pl.BlockSpec((1, 1), lambda i: (0, 0)),
        ],
        out_shape=[
            jax.ShapeDtypeStruct((_E, n_tokens), jnp.float32),
            jax.ShapeDtypeStruct((_TOPK, n_tokens), jnp.int32),
            jax.ShapeDtypeStruct((1, 1), jnp.float32),
        ],
        scratch_shapes=[pltpu.VMEM((_E, 2), jnp.float32)],
        compiler_params=pltpu.CompilerParams(
            dimension_semantics=("arbitrary",),
            vmem_limit_bytes=128 * 1024 * 1024,
        ),
    )(x, x, w, b, noise_t)

    return (probs_t.T.reshape(_B, _S, _E),
            idx_t.T.reshape(_B, _S, _TOPK),
            loss.reshape(()))
